# baseline (speedup 1.0000x reference)
import functools
import math

import jax
import jax.numpy as jnp
from jax import lax
from jax.experimental import pallas as pl
from jax.experimental.pallas import tpu as pltpu
from jax.experimental.pallas import tpu_sc as plsc

HEADS = 8
OUT = 16
DK = 32
KNN = 16
ATTN_TAU = 1.0
ATTN_GAMMA = 0.5
ATTN_MU = 0.7

SC_NC = 2
SC_NS = 16
SC_NW = SC_NC * SC_NS
SC_CH = 128
SC_GRP = 8

R1 = 2000
R3 = 1000


def _proj_body(wq_ref, wk_ref, a_ref):
    scale = 1.0 / math.sqrt(DK)
    for h in range(HEADS):
        ah = lax.dot_general(
            wq_ref[h], wk_ref[h], (((1,), (1,)), ((), ())),
            preferred_element_type=jnp.float32)
        a_ref[:, h * OUT:(h + 1) * OUT] = ah * scale


def _make_proj(WQ, WK):
    return pl.pallas_call(
        _proj_body,
        out_shape=jax.ShapeDtypeStruct((OUT, HEADS * OUT), jnp.float32),
    )(WQ, WK)


def _enc_body(x_ref, w0_ref, b0_ref, w1_ref, b1_ref, w2_ref, z_ref):
    xb = x_ref[...]
    h0 = jnp.maximum(
        jnp.dot(xb, w0_ref[...], preferred_element_type=jnp.float32)
        + b0_ref[...], 0.0)
    h1 = jnp.maximum(
        jnp.dot(h0, w1_ref[...], preferred_element_type=jnp.float32)
        + b1_ref[...], 0.0)
    z_ref[...] = jnp.dot(h1, w2_ref[...], preferred_element_type=jnp.float32)


def _encode(x, W0, b0, W1, b1, W2):
    n, d = x.shape
    grid = n // R1
    full = lambda i: (0, 0)
    return pl.pallas_call(
        _enc_body,
        grid=(grid,),
        in_specs=[
            pl.BlockSpec((R1, d), lambda i: (i, 0)),
            pl.BlockSpec(W0.shape, full),
            pl.BlockSpec((1, 64), full),
            pl.BlockSpec(W1.shape, full),
            pl.BlockSpec((1, 64), full),
            pl.BlockSpec(W2.shape, full),
        ],
        out_specs=pl.BlockSpec((R1, OUT), lambda i: (i, 0)),
        out_shape=jax.ShapeDtypeStruct((n, OUT), jnp.float32),
    )(x, W0, b0, W1, b1, W2)


def _gather_rows(table, idx2, groups_per_worker):
    chunks_pad = idx2.shape[0]
    half = groups_per_worker // 2
    mesh = plsc.VectorSubcoreMesh(
        core_axis_name="c", subcore_axis_name="s",
        num_cores=SC_NC, num_subcores=SC_NS)

    @functools.partial(
        pl.kernel,
        out_type=jax.ShapeDtypeStruct((chunks_pad, SC_CH, OUT), jnp.float32),
        mesh=mesh,
        scratch_types=[
            pltpu.VMEM((2, SC_GRP, SC_CH), jnp.int32),
            pltpu.VMEM((2, SC_GRP, SC_CH, OUT), jnp.float32),
            pltpu.SemaphoreType.DMA((2,)),
            pltpu.SemaphoreType.DMA((2,)),
            pltpu.SemaphoreType.DMA((2,)),
        ],
        compiler_params=pltpu.CompilerParams(use_tc_tiling_on_sc=False),
    )
    def gather_kernel(table_hbm, idx_hbm, out_hbm, idx_v, rows_v,
                      isem, gsem, osem):
        wid = lax.axis_index("s") * SC_NC + lax.axis_index("c")
        base = wid * groups_per_worker * SC_GRP
        last = base + (groups_per_worker - 1) * SC_GRP

        for b in range(2):
            pltpu.async_copy(idx_hbm.at[pl.ds(base + b * SC_GRP, SC_GRP)],
                             idx_v.at[b], isem.at[b])

        def body(i, carry):
            for b in range(2):
                g = 2 * i + b
                cstart = base + g * SC_GRP
                pltpu.make_async_copy(
                    idx_hbm.at[pl.ds(cstart, SC_GRP)], idx_v.at[b],
                    isem.at[b]).wait()

                @pl.when(i > 0)
                def _():
                    pltpu.make_async_copy(
                        rows_v.at[b], out_hbm.at[pl.ds(cstart, SC_GRP)],
                        osem.at[b]).wait()

                copies = [
                    pltpu.async_copy(table_hbm.at[idx_v.at[b].at[j]],
                                     rows_v.at[b].at[j], gsem.at[b])
                    for j in range(SC_GRP)
                ]
                for cp in copies:
                    cp.wait()

                cnext = jnp.minimum(cstart + 2 * SC_GRP, last)
                pltpu.async_copy(idx_hbm.at[pl.ds(cnext, SC_GRP)],
                                 idx_v.at[b], isem.at[b])
                pltpu.async_copy(rows_v.at[b],
                                 out_hbm.at[pl.ds(cstart, SC_GRP)],
                                 osem.at[b])
            return carry

        lax.fori_loop(0, half, body, 0)
        for b in range(2):
            pltpu.make_async_copy(
                idx_hbm.at[pl.ds(base, SC_GRP)], idx_v.at[b],
                isem.at[b]).wait()
            pltpu.make_async_copy(
                rows_v.at[b], out_hbm.at[pl.ds(base, SC_GRP)],
                osem.at[b]).wait()

    return gather_kernel(table, idx2)


def _att_body(zn2_ref, z_ref, a_ref, wb_ref, ct_ref, selk_ref, tile16_ref,
              selh2_ref, selh3_ref, eta_ref, out_ref):
    zn2 = zn2_ref[...]
    z = z_ref[...]
    q = jnp.dot(z, a_ref[...], preferred_element_type=jnp.float32)
    zc = z - ct_ref[:, :OUT]
    d2 = jnp.sum(zc * zc, axis=1, keepdims=True)
    selk = selk_ref[...]
    tile16 = tile16_ref[...]
    eta = eta_ref[0, 0]
    m = jnp.maximum(eta, 0.0) + jnp.log1p(jnp.exp(-jnp.abs(eta)))
    diff = zn2 - ct_ref[...]
    d2n = jnp.dot(diff * diff, selk, preferred_element_type=jnp.float32)
    fpos = jnp.maximum(d2 - m, 0.0)
    fposn = jnp.maximum(d2n - m, 0.0)
    damp = ATTN_GAMMA * (fpos + fposn)
    inv_tau = 1.0 / max(1e-6, ATTN_TAU)
    parts = []
    for h in range(HEADS):
        qh = q[:, h * OUT:(h + 1) * OUT]
        qrep = jnp.dot(qh, tile16, preferred_element_type=jnp.float32)
        lg = jnp.dot(zn2 * qrep, selk, preferred_element_type=jnp.float32)
        lg = (lg - damp) * inv_tau
        parts.append(lg - jnp.max(lg, axis=1, keepdims=True))
    lgs = jnp.concatenate(parts, axis=1)
    e = jnp.exp(lgs)
    denom = jnp.dot(e, selh2_ref[...], preferred_element_type=jnp.float32)
    wsum = jnp.dot(e / denom, selh3_ref[...],
                   preferred_element_type=jnp.float32)
    wattn = jnp.maximum(wsum * (1.0 / HEADS), 0.0)
    w = (1.0 - ATTN_MU) * wb_ref[...] + ATTN_MU * wattn
    wn = w / jnp.maximum(jnp.sum(w, axis=1, keepdims=True), 1e-8)
    dd = d2 - d2n
    smooth = jnp.sum(wn * dd * dd, axis=1, keepdims=True)
    out_ref[...] = jnp.concatenate([d2 - m, smooth], axis=1)


def _attention(znp2, Z, A, W_base, c, eta2):
    n = Z.shape[0]
    grid = n // R3
    ct = jnp.tile(c.reshape(1, OUT), (1, KNN))
    lane = jnp.arange(KNN * OUT, dtype=jnp.int32)
    selk = (lane[:, None] // OUT
            == jnp.arange(KNN, dtype=jnp.int32)[None, :]).astype(jnp.float32)
    tile16 = (jnp.arange(OUT, dtype=jnp.int32)[:, None]
              == lane[None, :] % OUT).astype(jnp.float32)
    hlane = jnp.arange(HEADS * KNN, dtype=jnp.int32)
    selh2 = (hlane[:, None] // KNN == hlane[None, :] // KNN).astype(
        jnp.float32)
    selh3 = (hlane[:, None] % KNN
             == jnp.arange(KNN, dtype=jnp.int32)[None, :]).astype(
        jnp.float32)
    full = lambda i: (0, 0)
    return pl.pallas_call(
        _att_body,
        grid=(grid,),
        in_specs=[
            pl.BlockSpec((R3, KNN * OUT), lambda i: (i, 0)),
            pl.BlockSpec((R3, OUT), lambda i: (i, 0)),
            pl.BlockSpec((OUT, HEADS * OUT), full),
            pl.BlockSpec((R3, KNN), lambda i: (i, 0)),
            pl.BlockSpec((1, KNN * OUT), full),
            pl.BlockSpec((KNN * OUT, KNN), full),
            pl.BlockSpec((OUT, KNN * OUT), full),
            pl.BlockSpec((HEADS * KNN, HEADS * KNN), full),
            pl.BlockSpec((HEADS * KNN, KNN), full),
            pl.BlockSpec((1, 1), full),
        ],
        out_specs=pl.BlockSpec((R3, 2), lambda i: (i, 0)),
        out_shape=jax.ShapeDtypeStruct((n, 2), jnp.float32),
    )(znp2, Z, A, W_base, ct, selk, tile16, selh2, selh3, eta2)


def kernel(x, nn_idx, W_base, W0, b0, W1, b1, W2, WQ, WK, c, eta):
    n = x.shape[0]
    A = _make_proj(WQ, WK)
    Z = _encode(x, W0, b0.reshape(1, -1), W1, b1.reshape(1, -1), W2)

    e = n * KNN
    chunks = -(-e // SC_CH)
    gpw = -(-chunks // (SC_NW * SC_GRP))
    gpw += gpw % 2
    chunks_pad = SC_NW * gpw * SC_GRP
    flat = nn_idx.reshape(-1)
    flat = jnp.pad(flat, (0, chunks_pad * SC_CH - e))
    idx2 = flat.reshape(chunks_pad, SC_CH)

    znei = _gather_rows(Z, idx2, gpw)
    znp2 = znei.reshape(-1, KNN * OUT)

    return _attention(znp2, Z, A, W_base, c, eta.reshape(1, 1))


# speedup vs baseline: 1.6710x; 1.6710x over previous
import functools
import math

import jax
import jax.numpy as jnp
from jax import lax
from jax.experimental import pallas as pl
from jax.experimental.pallas import tpu as pltpu
from jax.experimental.pallas import tpu_sc as plsc

HEADS = 8
OUT = 16
DK = 32
KNN = 16
ATTN_TAU = 1.0
ATTN_GAMMA = 0.5
ATTN_MU = 0.7

SC_NC = 2
SC_NS = 16
SC_NW = SC_NC * SC_NS
SC_CH = 128
SC_GRP = 8

R1 = 2000
R3 = 1000


def _proj_body(wq_ref, wk_ref, a_ref):
    scale = 1.0 / math.sqrt(DK)
    for h in range(HEADS):
        ah = lax.dot_general(
            wq_ref[h], wk_ref[h], (((1,), (1,)), ((), ())),
            preferred_element_type=jnp.float32)
        a_ref[:, h * OUT:(h + 1) * OUT] = ah * scale


def _make_proj(WQ, WK):
    return pl.pallas_call(
        _proj_body,
        out_shape=jax.ShapeDtypeStruct((OUT, HEADS * OUT), jnp.float32),
    )(WQ, WK)


def _enc_body(x_ref, w0_ref, b0_ref, w1_ref, b1_ref, w2_ref, z_ref):
    xb = x_ref[...]
    h0 = jnp.maximum(
        jnp.dot(xb, w0_ref[...], preferred_element_type=jnp.float32)
        + b0_ref[...], 0.0)
    h1 = jnp.maximum(
        jnp.dot(h0, w1_ref[...], preferred_element_type=jnp.float32)
        + b1_ref[...], 0.0)
    z_ref[...] = jnp.dot(h1, w2_ref[...], preferred_element_type=jnp.float32)


def _encode(x, W0, b0, W1, b1, W2):
    n, d = x.shape
    grid = n // R1
    full = lambda i: (0, 0)
    return pl.pallas_call(
        _enc_body,
        grid=(grid,),
        in_specs=[
            pl.BlockSpec((R1, d), lambda i: (i, 0)),
            pl.BlockSpec(W0.shape, full),
            pl.BlockSpec((1, 64), full),
            pl.BlockSpec(W1.shape, full),
            pl.BlockSpec((1, 64), full),
            pl.BlockSpec(W2.shape, full),
        ],
        out_specs=pl.BlockSpec((R1, OUT), lambda i: (i, 0)),
        out_shape=jax.ShapeDtypeStruct((n, OUT), jnp.float32),
    )(x, W0, b0, W1, b1, W2)


def _gather_rows(table, idx2, groups_per_worker):
    chunks_pad = idx2.shape[0]
    half = groups_per_worker // 2
    mesh = plsc.VectorSubcoreMesh(
        core_axis_name="c", subcore_axis_name="s",
        num_cores=SC_NC, num_subcores=SC_NS)

    @functools.partial(
        pl.kernel,
        out_type=jax.ShapeDtypeStruct((chunks_pad, SC_CH, OUT), jnp.float32),
        mesh=mesh,
        scratch_types=[
            pltpu.VMEM((2, SC_GRP, SC_CH), jnp.int32),
            pltpu.VMEM((2, SC_GRP, SC_CH, OUT), jnp.float32),
            pltpu.SemaphoreType.DMA((2,)),
            pltpu.SemaphoreType.DMA((2,)),
            pltpu.SemaphoreType.DMA((2,)),
        ],
        compiler_params=pltpu.CompilerParams(use_tc_tiling_on_sc=False),
    )
    def gather_kernel(table_hbm, idx_hbm, out_hbm, idx_v, rows_v,
                      isem, gsem, osem):
        wid = lax.axis_index("s") * SC_NC + lax.axis_index("c")
        base = wid * groups_per_worker * SC_GRP
        last = base + (groups_per_worker - 1) * SC_GRP

        for b in range(2):
            pltpu.async_copy(idx_hbm.at[pl.ds(base + b * SC_GRP, SC_GRP)],
                             idx_v.at[b], isem.at[b])

        def body(i, carry):
            for b in range(2):
                g = 2 * i + b
                cstart = base + g * SC_GRP
                pltpu.make_async_copy(
                    idx_hbm.at[pl.ds(cstart, SC_GRP)], idx_v.at[b],
                    isem.at[b]).wait()

                @pl.when(i > 0)
                def _():
                    pltpu.make_async_copy(
                        rows_v.at[b], out_hbm.at[pl.ds(cstart, SC_GRP)],
                        osem.at[b]).wait()

                copies = [
                    pltpu.async_copy(table_hbm.at[idx_v.at[b].at[j]],
                                     rows_v.at[b].at[j], gsem.at[b])
                    for j in range(SC_GRP)
                ]
                for cp in copies:
                    cp.wait()

                cnext = jnp.minimum(cstart + 2 * SC_GRP, last)
                pltpu.async_copy(idx_hbm.at[pl.ds(cnext, SC_GRP)],
                                 idx_v.at[b], isem.at[b])
                pltpu.async_copy(rows_v.at[b],
                                 out_hbm.at[pl.ds(cstart, SC_GRP)],
                                 osem.at[b])
            return carry

        lax.fori_loop(0, half, body, 0)
        for b in range(2):
            pltpu.make_async_copy(
                idx_hbm.at[pl.ds(base, SC_GRP)], idx_v.at[b],
                isem.at[b]).wait()
            pltpu.make_async_copy(
                rows_v.at[b], out_hbm.at[pl.ds(base, SC_GRP)],
                osem.at[b]).wait()

    return gather_kernel(table, idx2)


def _att_body(zn2_ref, z_ref, a_ref, wb_ref, ct_ref, selk_ref, tile16_ref,
              eta_ref, out_ref):
    zn2 = zn2_ref[...]
    z = z_ref[...]
    q = jnp.dot(z, a_ref[...], preferred_element_type=jnp.float32)
    zc = z - ct_ref[:, :OUT]
    d2 = jnp.sum(zc * zc, axis=1, keepdims=True)
    selk = selk_ref[...]
    tile16 = tile16_ref[...]
    eta = eta_ref[0, 0]
    m = jnp.maximum(eta, 0.0) + jnp.log1p(jnp.exp(-jnp.abs(eta)))
    diff = zn2 - ct_ref[...]
    d2n = jnp.dot(diff * diff, selk, preferred_element_type=jnp.float32)
    fpos = jnp.maximum(d2 - m, 0.0)
    fposn = jnp.maximum(d2n - m, 0.0)
    damp = ATTN_GAMMA * (fpos + fposn)
    inv_tau = 1.0 / max(1e-6, ATTN_TAU)
    acc = jnp.zeros_like(damp)
    for h in range(HEADS):
        qh = q[:, h * OUT:(h + 1) * OUT]
        qrep = jnp.dot(qh, tile16, preferred_element_type=jnp.float32)
        lg = jnp.dot(zn2 * qrep, selk, preferred_element_type=jnp.float32)
        lg = (lg - damp) * inv_tau
        lg = lg - jnp.max(lg, axis=1, keepdims=True)
        e = jnp.exp(lg)
        acc = acc + e / jnp.sum(e, axis=1, keepdims=True)
    wattn = jnp.maximum(acc * (1.0 / HEADS), 0.0)
    w = (1.0 - ATTN_MU) * wb_ref[...] + ATTN_MU * wattn
    wn = w / jnp.maximum(jnp.sum(w, axis=1, keepdims=True), 1e-8)
    dd = d2 - d2n
    smooth = jnp.sum(wn * dd * dd, axis=1, keepdims=True)
    out_ref[...] = jnp.concatenate([d2 - m, smooth], axis=1)


def _attention(znp2, Z, A, W_base, c, eta2):
    n = Z.shape[0]
    grid = n // R3
    ct = jnp.tile(c.reshape(1, OUT), (1, KNN))
    lane = jnp.arange(KNN * OUT, dtype=jnp.int32)
    selk = (lane[:, None] // OUT
            == jnp.arange(KNN, dtype=jnp.int32)[None, :]).astype(jnp.float32)
    tile16 = (jnp.arange(OUT, dtype=jnp.int32)[:, None]
              == lane[None, :] % OUT).astype(jnp.float32)
    full = lambda i: (0, 0)
    return pl.pallas_call(
        _att_body,
        grid=(grid,),
        in_specs=[
            pl.BlockSpec((R3, KNN * OUT), lambda i: (i, 0)),
            pl.BlockSpec((R3, OUT), lambda i: (i, 0)),
            pl.BlockSpec((OUT, HEADS * OUT), full),
            pl.BlockSpec((R3, KNN), lambda i: (i, 0)),
            pl.BlockSpec((1, KNN * OUT), full),
            pl.BlockSpec((KNN * OUT, KNN), full),
            pl.BlockSpec((OUT, KNN * OUT), full),
            pl.BlockSpec((1, 1), full),
        ],
        out_specs=pl.BlockSpec((R3, 2), lambda i: (i, 0)),
        out_shape=jax.ShapeDtypeStruct((n, 2), jnp.float32),
    )(znp2, Z, A, W_base, ct, selk, tile16, eta2)


def kernel(x, nn_idx, W_base, W0, b0, W1, b1, W2, WQ, WK, c, eta):
    n = x.shape[0]
    A = _make_proj(WQ, WK)
    Z = _encode(x, W0, b0.reshape(1, -1), W1, b1.reshape(1, -1), W2)

    e = n * KNN
    chunks = -(-e // SC_CH)
    gpw = -(-chunks // (SC_NW * SC_GRP))
    gpw += gpw % 2
    chunks_pad = SC_NW * gpw * SC_GRP
    flat = nn_idx.reshape(-1)
    flat = jnp.pad(flat, (0, chunks_pad * SC_CH - e))
    idx2 = flat.reshape(chunks_pad, SC_CH)

    znei = _gather_rows(Z, idx2, gpw)
    znp2 = znei.reshape(-1, KNN * OUT)

    return _attention(znp2, Z, A, W_base, c, eta.reshape(1, 1))
